# Initial kernel scaffold; baseline (speedup 1.0000x reference)
import jax
import jax.numpy as jnp
from jax.experimental import pallas as pl


def kernel(x, edge_index, edge_attr, direct_temporal_mask, gap_temporal_mask, proximity_mask, W_t, att_src_t, att_dst_t, W_e_t, att_e_t, b_t, W_g, att_src_g, att_dst_g, W_e_g, att_e_g, b_g, W_p, att_src_p, att_dst_p, W_e_p, att_e_p, b_p, W_fus, b_fus, W2, att_src2, att_dst2, b2, W3, att_src3, att_dst3, b3, W_c1, b_c1, W_c2, b_c2):
    raise NotImplementedError("write your pallas kernel here")


# speedup vs baseline: 2.5309x; 2.5309x over previous
import jax
import jax.numpy as jnp
from jax.experimental import pallas as pl

_N = 50000
_E = 1600000
_H = 4
_C1 = 10
_HID = 128
_C2 = _HID // _H
_NC = 5


def _lrelu(v):
    return jnp.where(v >= 0, v, 0.2 * v)


def _mlp_body(h_ref, w1_ref, b1_ref, w2_ref, b2_ref, o_ref):
    h = h_ref[...]
    z = jnp.maximum(jnp.dot(h, w1_ref[...], preferred_element_type=jnp.float32)
                    + b1_ref[...], 0.0)
    lg = jnp.dot(z, w2_ref[...], preferred_element_type=jnp.float32) + b2_ref[...]
    m = jnp.max(lg, axis=-1, keepdims=True)
    e = jnp.exp(lg - m)
    o_ref[...] = lg - m - jnp.log(jnp.sum(e, axis=-1, keepdims=True))


def _mlp_head(h, W_c1, b_c1, W_c2, b_c2):
    blk = 1000
    grid = (_N // blk,)
    return pl.pallas_call(
        _mlp_body,
        grid=grid,
        in_specs=[
            pl.BlockSpec((blk, _HID), lambda i: (i, 0)),
            pl.BlockSpec((_HID, _HID // 2), lambda i: (0, 0)),
            pl.BlockSpec((1, _HID // 2), lambda i: (0, 0)),
            pl.BlockSpec((_HID // 2, _NC), lambda i: (0, 0)),
            pl.BlockSpec((1, _NC), lambda i: (0, 0)),
        ],
        out_specs=pl.BlockSpec((blk, _NC), lambda i: (i, 0)),
        out_shape=jax.ShapeDtypeStruct((_N, _NC), jnp.float32),
    )(h, W_c1, b_c1.reshape(1, -1), W_c2, b_c2.reshape(1, -1))


def _branch(xf, xs, xd, src, dst, edge_attr, mask, W, att_s, att_d, W_e, att_e, b):
    Wm = W.reshape(_H, _C1)
    ws = (Wm * att_s).sum(-1)
    wd = (Wm * att_d).sum(-1)
    v = (W_e.reshape(3, _H, _C1) * att_e[None]).sum(-1)
    ae = edge_attr @ v
    alpha = _lrelu(xs[:, None] * ws + xd[:, None] * wd + ae)
    xmax = jnp.max(xf)
    xmin = jnp.min(xf)
    ms = jnp.maximum(xmax * ws, xmin * ws)
    md = jnp.maximum(xmax * wd, xmin * wd)
    M = _lrelu(ms + md + jnp.max(ae, axis=0))
    m = mask.astype(jnp.float32)
    ex = m[:, None] * jnp.exp(alpha - M)
    den = jax.ops.segment_sum(ex, dst, num_segments=_N)
    num = jax.ops.segment_sum(ex * xs[:, None], dst, num_segments=_N)
    cnt = jax.ops.segment_sum(m, dst, num_segments=_N)
    easum = jax.ops.segment_sum(edge_attr * m[:, None], dst, num_segments=_N)
    loop_attr = easum / jnp.maximum(cnt, 1.0)[:, None]
    alpha_loop = _lrelu(xf[:, None] * (ws + wd) + loop_attr @ v)
    exloop = jnp.exp(alpha_loop - M)
    dent = den + exloop
    numt = num + exloop * xf[:, None]
    coef = numt / dent
    out = coef[:, :, None] * Wm[None]
    return out.reshape(_N, _H * _C1) + b


def _gat_dense(h, src, dst, W, att_s, att_d, b):
    g = h @ W
    g3 = g.reshape(_N, _H, _C2)
    asrc = (g3 * att_s).sum(-1)
    adst = (g3 * att_d).sum(-1)
    alpha = _lrelu(asrc[src] + adst[dst])
    M = _lrelu(jnp.max(asrc, axis=0) + jnp.max(adst, axis=0))
    ex = jnp.exp(alpha - M)
    den = jax.ops.segment_sum(ex, dst, num_segments=_N)
    msg = jax.ops.segment_sum(ex[:, :, None] * g3[src], dst, num_segments=_N)
    exloop = jnp.exp(_lrelu(asrc + adst) - M)
    dent = den + exloop
    out = (msg + exloop[:, :, None] * g3) / dent[:, :, None]
    return out.reshape(_N, _HID) + b


def kernel(x, edge_index, edge_attr, direct_temporal_mask, gap_temporal_mask,
           proximity_mask, W_t, att_src_t, att_dst_t, W_e_t, att_e_t, b_t,
           W_g, att_src_g, att_dst_g, W_e_g, att_e_g, b_g,
           W_p, att_src_p, att_dst_p, W_e_p, att_e_p, b_p,
           W_fus, b_fus, W2, att_src2, att_dst2, b2,
           W3, att_src3, att_dst3, b3, W_c1, b_c1, W_c2, b_c2):
    src, dst = edge_index[0], edge_index[1]
    xf = x[:, 0]
    xs = xf[src]
    xd = xf[dst]
    h_t = _branch(xf, xs, xd, src, dst, edge_attr, direct_temporal_mask,
                  W_t, att_src_t, att_dst_t, W_e_t, att_e_t, b_t)
    h_g = _branch(xf, xs, xd, src, dst, edge_attr, gap_temporal_mask,
                  W_g, att_src_g, att_dst_g, W_e_g, att_e_g, b_g)
    h_p = _branch(xf, xs, xd, src, dst, edge_attr, proximity_mask,
                  W_p, att_src_p, att_dst_p, W_e_p, att_e_p, b_p)
    h = jnp.concatenate([h_t, h_g, h_p], axis=-1)
    h = jax.nn.relu(h @ W_fus + b_fus)
    h = jax.nn.relu(_gat_dense(h, src, dst, W2, att_src2, att_dst2, b2))
    h = jax.nn.relu(_gat_dense(h, src, dst, W3, att_src3, att_dst3, b3))
    return _mlp_head(h, W_c1, b_c1, W_c2, b_c2)


# speedup vs baseline: 12.5221x; 4.9477x over previous
import functools

import jax
import jax.numpy as jnp
from jax import lax
from jax.experimental import pallas as pl
from jax.experimental.pallas import tpu as pltpu
from jax.experimental.pallas import tpu_sc as plsc

_N = 50000
_E = 1600000
_H = 4
_C1 = 10
_HID = 128
_C2 = _HID // _H
_NC = 5

_NW = 32
_EPW = _E // _NW
_EK = 80
_NCH = _EPW // _EK
_MK = 64
_MCH = _EPW // _MK

_mesh = plsc.VectorSubcoreMesh(core_axis_name="c", subcore_axis_name="s")
_sc_params = pltpu.CompilerParams(needs_layout_passes=False,
                                  use_tc_tiling_on_sc=False)


def _lrelu(v):
    return jnp.where(v >= 0, v, 0.2 * v)


def _msg8_body(src_hbm, dst_hbm, srcblk_hbm, asrc_hbm, adst_hbm, m_hbm, g_hbm,
               zero_hbm, out_hbm, asrc_v, adst_v, m_v, src_v, dst_v, idx_v,
               grow_v, rows_v, acc_sh, sem):
    cid = lax.axis_index("c")
    sid = lax.axis_index("s")
    wid = sid * 2 + cid
    base = wid * (_MCH * _MK)
    lane = jnp.arange(16, dtype=jnp.int32)
    col_den = jnp.full((16,), 8, jnp.int32)
    zv = jnp.zeros((16,), jnp.float32)

    def phase(ph, carry):
        h = ph // 5
        pltpu.sync_copy(asrc_hbm.at[h], asrc_v)
        pltpu.sync_copy(adst_hbm.at[h], adst_v)
        pltpu.sync_copy(m_hbm.at[h], m_v)

        @pl.when(sid == 0)
        def _():
            pltpu.sync_copy(zero_hbm, acc_sh)

        plsc.subcore_barrier()

        def chunk_at(off):
            pltpu.sync_copy(src_hbm.at[pl.ds(off, _MK)], src_v)
            pltpu.sync_copy(dst_hbm.at[pl.ds(off, _MK)], dst_v)
            pltpu.sync_copy(srcblk_hbm.at[ph, pl.ds(off, _MK)], idx_v)
            pltpu.async_copy(g_hbm.at[idx_v], grow_v, sem).wait()
            mv = m_v[...]
            for g in range(_MK // 16):
                s16 = src_v[pl.ds(g * 16, 16)]
                d16 = dst_v[pl.ds(g * 16, 16)]
                a = (plsc.load_gather(asrc_v, [s16])
                     + plsc.load_gather(adst_v, [d16]))
                ex = jnp.exp(_lrelu(a) - mv)
                l16 = g * 16 + lane
                for c in range(8):
                    cc = jnp.full((16,), c, jnp.int32)
                    gv = plsc.load_gather(grow_v, [l16, cc])
                    plsc.store_scatter(rows_v, [l16, cc], ex * gv)
            pltpu.sync_copy(rows_v, acc_sh.at[dst_v], add=True)

        def chunk(i, c2):
            chunk_at(pl.multiple_of(base + i * _MK, 8))
            return c2

        lax.fori_loop(0, _MCH, chunk, 0)

        @pl.when(wid < 8)
        def _():
            chunk_at(pl.multiple_of(_NW * _MCH * _MK + wid * _MK, 8))

        plsc.subcore_barrier()

        @pl.when(sid == 0)
        def _():
            pltpu.sync_copy(acc_sh, out_hbm.at[cid, ph])

        return carry

    lax.fori_loop(0, 20, phase, 0)


_msg8_pass = functools.partial(
    pl.kernel,
    _msg8_body,
    out_type=jax.ShapeDtypeStruct((2, 20, _N, 8), jnp.float32),
    mesh=_mesh,
    scratch_types=[
        pltpu.VMEM((_N,), jnp.float32),
        pltpu.VMEM((_N,), jnp.float32),
        pltpu.VMEM((16,), jnp.float32),
        pltpu.VMEM((_MK,), jnp.int32),
        pltpu.VMEM((_MK,), jnp.int32),
        pltpu.VMEM((_MK,), jnp.int32),
        pltpu.VMEM((_MK, 8), jnp.float32),
        pltpu.VMEM((_MK, 8), jnp.float32),
        pltpu.VMEM_SHARED((_N, 8), jnp.float32),
        pltpu.SemaphoreType.DMA,
    ],
    compiler_params=_sc_params,
)()


def _branch3_body(src_hbm, dst_hbm, x_hbm, c_hbm, ea0_hbm, ea1_hbm, ea2_hbm,
                  m_hbm, zero_hbm, out_hbm, x_v, c_v, src_v, dst_v, ea0_v,
                  ea1_v, ea2_v, m_v, rows_v, acc_sh):
    cid = lax.axis_index("c")
    sid = lax.axis_index("s")
    wid = sid * 2 + cid
    base = wid * _EPW
    lane = jnp.arange(16, dtype=jnp.int32)

    pltpu.sync_copy(x_hbm, x_v)
    for b in range(3):
        pltpu.sync_copy(c_hbm.at[b], c_v)

        @pl.when(sid == 0)
        def _():
            pltpu.sync_copy(zero_hbm, acc_sh)

        plsc.subcore_barrier()

        def chunk(i, carry):
            off = pl.multiple_of(base + i * _EK, 8)
            pltpu.sync_copy(src_hbm.at[pl.ds(off, _EK)], src_v)
            pltpu.sync_copy(dst_hbm.at[pl.ds(off, _EK)], dst_v)
            pltpu.sync_copy(ea0_hbm.at[pl.ds(off, _EK)], ea0_v)
            pltpu.sync_copy(ea1_hbm.at[pl.ds(off, _EK)], ea1_v)
            pltpu.sync_copy(ea2_hbm.at[pl.ds(off, _EK)], ea2_v)
            pltpu.sync_copy(m_hbm.at[b, pl.ds(off, _EK)], m_v)
            for g in range(_EK // 16):
                sl = pl.ds(g * 16, 16)
                s16 = src_v[sl]
                d16 = dst_v[sl]
                xs = plsc.load_gather(x_v, [s16])
                xd = plsc.load_gather(x_v, [d16])
                e0 = ea0_v[sl]
                e1 = ea1_v[sl]
                e2 = ea2_v[sl]
                m = m_v[sl]
                l16 = g * 16 + lane
                for h in range(4):
                    ws = c_v[pl.ds(h * 16, 16)]
                    wd = c_v[pl.ds((4 + h) * 16, 16)]
                    mh = c_v[pl.ds((8 + h) * 16, 16)]
                    v0 = c_v[pl.ds((12 + h) * 16, 16)]
                    v1 = c_v[pl.ds((16 + h) * 16, 16)]
                    v2 = c_v[pl.ds((20 + h) * 16, 16)]
                    z = xs * ws + xd * wd + e0 * v0 + e1 * v1 + e2 * v2
                    ex = m * jnp.exp(_lrelu(z) - mh)
                    plsc.store_scatter(
                        rows_v, [l16, jnp.full((16,), h, jnp.int32)], ex)
                    plsc.store_scatter(
                        rows_v, [l16, jnp.full((16,), 4 + h, jnp.int32)], ex * xs)
                plsc.store_scatter(rows_v, [l16, jnp.full((16,), 8, jnp.int32)], m)
                plsc.store_scatter(rows_v, [l16, jnp.full((16,), 9, jnp.int32)], e0 * m)
                plsc.store_scatter(rows_v, [l16, jnp.full((16,), 10, jnp.int32)], e1 * m)
                plsc.store_scatter(rows_v, [l16, jnp.full((16,), 11, jnp.int32)], e2 * m)
            pltpu.sync_copy(rows_v, acc_sh.at[dst_v], add=True)
            return carry

        lax.fori_loop(0, _NCH, chunk, 0)
        plsc.subcore_barrier()

        @pl.when(sid == 0)
        def _():
            pltpu.sync_copy(acc_sh, out_hbm.at[cid, b])


_branch3_pass = functools.partial(
    pl.kernel,
    _branch3_body,
    out_type=jax.ShapeDtypeStruct((2, 3, _N, 12), jnp.float32),
    mesh=_mesh,
    scratch_types=[
        pltpu.VMEM((_N,), jnp.float32),
        pltpu.VMEM((24 * 16,), jnp.float32),
        pltpu.VMEM((_EK,), jnp.int32),
        pltpu.VMEM((_EK,), jnp.int32),
        pltpu.VMEM((_EK,), jnp.float32),
        pltpu.VMEM((_EK,), jnp.float32),
        pltpu.VMEM((_EK,), jnp.float32),
        pltpu.VMEM((_EK,), jnp.float32),
        pltpu.VMEM((_EK, 12), jnp.float32),
        pltpu.VMEM_SHARED((_N, 12), jnp.float32),
    ],
    compiler_params=_sc_params,
)()


def _gproj_body(h_ref, w_ref, as_ref, ad_ref, g_ref, asrc_ref, adst_ref):
    g = jnp.dot(h_ref[...], w_ref[...], preferred_element_type=jnp.float32)
    g_ref[...] = g
    asrc_ref[...] = jnp.dot(g, as_ref[...], preferred_element_type=jnp.float32)
    adst_ref[...] = jnp.dot(g, ad_ref[...], preferred_element_type=jnp.float32)


def _gproj(h, W, As, Ad):
    blk = 1000
    return pl.pallas_call(
        _gproj_body,
        grid=(_N // blk,),
        in_specs=[
            pl.BlockSpec((blk, _HID), lambda i: (i, 0)),
            pl.BlockSpec((_HID, _HID), lambda i: (0, 0)),
            pl.BlockSpec((_HID, _H), lambda i: (0, 0)),
            pl.BlockSpec((_HID, _H), lambda i: (0, 0)),
        ],
        out_specs=[
            pl.BlockSpec((blk, _HID), lambda i: (i, 0)),
            pl.BlockSpec((blk, _H), lambda i: (i, 0)),
            pl.BlockSpec((blk, _H), lambda i: (i, 0)),
        ],
        out_shape=[
            jax.ShapeDtypeStruct((_N, _HID), jnp.float32),
            jax.ShapeDtypeStruct((_N, _H), jnp.float32),
            jax.ShapeDtypeStruct((_N, _H), jnp.float32),
        ],
    )(h, W, As, Ad)


def _fuse_body(a_ref, b_ref, c_ref, w1_ref, w2_ref, w3_ref, bias_ref, o_ref):
    z = (jnp.dot(a_ref[...], w1_ref[...], preferred_element_type=jnp.float32)
         + jnp.dot(b_ref[...], w2_ref[...], preferred_element_type=jnp.float32)
         + jnp.dot(c_ref[...], w3_ref[...], preferred_element_type=jnp.float32)
         + bias_ref[...])
    o_ref[...] = jnp.maximum(z, 0.0)


def _fuse(a, b, c, W_fus, b_fus):
    blk = 1000
    hc = _H * _C1
    return pl.pallas_call(
        _fuse_body,
        grid=(_N // blk,),
        in_specs=[
            pl.BlockSpec((blk, hc), lambda i: (i, 0)),
            pl.BlockSpec((blk, hc), lambda i: (i, 0)),
            pl.BlockSpec((blk, hc), lambda i: (i, 0)),
            pl.BlockSpec((hc, _HID), lambda i: (0, 0)),
            pl.BlockSpec((hc, _HID), lambda i: (0, 0)),
            pl.BlockSpec((hc, _HID), lambda i: (0, 0)),
            pl.BlockSpec((1, _HID), lambda i: (0, 0)),
        ],
        out_specs=pl.BlockSpec((blk, _HID), lambda i: (i, 0)),
        out_shape=jax.ShapeDtypeStruct((_N, _HID), jnp.float32),
    )(a, b, c, W_fus[:hc], W_fus[hc:2 * hc], W_fus[2 * hc:], b_fus.reshape(1, -1))


def _mlp_body(h_ref, w1_ref, b1_ref, w2_ref, b2_ref, o_ref):
    h = h_ref[...]
    z = jnp.maximum(jnp.dot(h, w1_ref[...], preferred_element_type=jnp.float32)
                    + b1_ref[...], 0.0)
    lg = jnp.dot(z, w2_ref[...], preferred_element_type=jnp.float32) + b2_ref[...]
    m = jnp.max(lg, axis=-1, keepdims=True)
    e = jnp.exp(lg - m)
    o_ref[...] = lg - m - jnp.log(jnp.sum(e, axis=-1, keepdims=True))


def _mlp_head(h, W_c1, b_c1, W_c2, b_c2):
    blk = 1000
    return pl.pallas_call(
        _mlp_body,
        grid=(_N // blk,),
        in_specs=[
            pl.BlockSpec((blk, _HID), lambda i: (i, 0)),
            pl.BlockSpec((_HID, _HID // 2), lambda i: (0, 0)),
            pl.BlockSpec((1, _HID // 2), lambda i: (0, 0)),
            pl.BlockSpec((_HID // 2, _NC), lambda i: (0, 0)),
            pl.BlockSpec((1, _NC), lambda i: (0, 0)),
        ],
        out_specs=pl.BlockSpec((blk, _NC), lambda i: (i, 0)),
        out_shape=jax.ShapeDtypeStruct((_N, _NC), jnp.float32),
    )(h, W_c1, b_c1.reshape(1, -1), W_c2, b_c2.reshape(1, -1))


def _branch_params(stats, W, att_s, att_d, W_e, att_e):
    xmax, xmin, eamax, eamin = stats
    Wm = W.reshape(_H, _C1)
    ws = (Wm * att_s).sum(-1)
    wd = (Wm * att_d).sum(-1)
    v = (W_e.reshape(3, _H, _C1) * att_e[None]).sum(-1)
    ms = jnp.maximum(xmax * ws, xmin * ws)
    md = jnp.maximum(xmax * wd, xmin * wd)
    aeb = jnp.maximum(eamax[:, None] * v, eamin[:, None] * v).sum(0)
    M = _lrelu(ms + md + aeb)
    consts = jnp.concatenate([ws, wd, M, v.reshape(12)])
    c_arr = jnp.tile(consts[:, None], (1, 16)).reshape(-1)
    return ws, wd, v, M, c_arr


def _branch_finish(xf, tot, ws, wd, v, M, W, b):
    den = tot[:, 0:4]
    num = tot[:, 4:8]
    cnt = tot[:, 8]
    eas = tot[:, 9:12]
    loop_attr = eas / jnp.maximum(cnt, 1.0)[:, None]
    alpha_loop = _lrelu(xf[:, None] * (ws + wd) + loop_attr @ v)
    exloop = jnp.exp(alpha_loop - M)
    coef = (num + exloop * xf[:, None]) / (den + exloop)
    out = coef[:, :, None] * W.reshape(_H, _C1)[None]
    return out.reshape(_N, _H * _C1) + b


def _gat_dense(h, src, dst, srcblk, zero8, W, att_s, att_d, b):
    emb = jnp.zeros((_HID, _H), jnp.float32)
    rows = jnp.arange(_HID)
    hsel = jnp.repeat(jnp.arange(_H), _C2)
    As = emb.at[rows, hsel].set(att_s.reshape(-1))
    Ad = emb.at[rows, hsel].set(att_d.reshape(-1))
    g, asrc, adst = _gproj(h, W, As, Ad)
    M = _lrelu(jnp.max(asrc, axis=0) + jnp.max(adst, axis=0))
    g4 = g.reshape(_N, 4, 32)
    gx = jnp.concatenate([g4, jnp.ones((_N, 4, 1), jnp.float32),
                          jnp.zeros((_N, 4, 7), jnp.float32)], axis=2)
    g_flat = jnp.transpose(gx.reshape(_N, 4, 5, 8),
                           (1, 2, 0, 3)).reshape(20 * _N, 8)
    m4 = jnp.tile(M[:, None], (1, 16))
    parts = _msg8_pass(src, dst, srcblk, asrc.T, adst.T, m4, g_flat, zero8)
    tot = (parts[0] + parts[1]).reshape(4, 5, _N, 8)
    msg = jnp.transpose(tot[:, :4], (2, 0, 1, 3)).reshape(_N, _H, _C2)
    den = tot[:, 4, :, 0].T
    exloop = jnp.exp(_lrelu(asrc + adst) - M)
    g3 = g.reshape(_N, _H, _C2)
    out = (msg + exloop[:, :, None] * g3) / (den + exloop)[:, :, None]
    return out.reshape(_N, _HID) + b


def kernel(x, edge_index, edge_attr, direct_temporal_mask, gap_temporal_mask,
           proximity_mask, W_t, att_src_t, att_dst_t, W_e_t, att_e_t, b_t,
           W_g, att_src_g, att_dst_g, W_e_g, att_e_g, b_g,
           W_p, att_src_p, att_dst_p, W_e_p, att_e_p, b_p,
           W_fus, b_fus, W2, att_src2, att_dst2, b2,
           W3, att_src3, att_dst3, b3, W_c1, b_c1, W_c2, b_c2):
    src = edge_index[0]
    dst = edge_index[1]
    xf = x[:, 0]
    eaT = edge_attr.T
    ea0, ea1, ea2 = eaT[0], eaT[1], eaT[2]
    masks = jnp.stack([direct_temporal_mask.astype(jnp.float32),
                       gap_temporal_mask.astype(jnp.float32),
                       proximity_mask.astype(jnp.float32)])
    zero12 = jnp.zeros((_N, 12), jnp.float32)
    zero8 = jnp.zeros((_N, 8), jnp.float32)
    stats = (jnp.max(xf), jnp.min(xf),
             jnp.max(edge_attr, axis=0), jnp.min(edge_attr, axis=0))
    p_t = _branch_params(stats, W_t, att_src_t, att_dst_t, W_e_t, att_e_t)
    p_g = _branch_params(stats, W_g, att_src_g, att_dst_g, W_e_g, att_e_g)
    p_p = _branch_params(stats, W_p, att_src_p, att_dst_p, W_e_p, att_e_p)
    c_all = jnp.stack([p_t[4], p_g[4], p_p[4]])
    parts = _branch3_pass(src, dst, xf, c_all, ea0, ea1, ea2, masks, zero12)
    tots = parts[0] + parts[1]
    h_t = _branch_finish(xf, tots[0], *p_t[:4], W_t, b_t)
    h_g = _branch_finish(xf, tots[1], *p_g[:4], W_g, b_g)
    h_p = _branch_finish(xf, tots[2], *p_p[:4], W_p, b_p)
    h = _fuse(h_t, h_g, h_p, W_fus, b_fus)
    srcblk = src[None, :] + (jnp.arange(20, dtype=jnp.int32) * _N)[:, None]
    h = jax.nn.relu(_gat_dense(h, src, dst, srcblk, zero8,
                               W2, att_src2, att_dst2, b2))
    h = jax.nn.relu(_gat_dense(h, src, dst, srcblk, zero8,
                               W3, att_src3, att_dst3, b3))
    return _mlp_head(h, W_c1, b_c1, W_c2, b_c2)


# speedup vs baseline: 28.7920x; 2.2993x over previous
import functools

import jax
import jax.numpy as jnp
from jax import lax
from jax.experimental import pallas as pl
from jax.experimental.pallas import tpu as pltpu
from jax.experimental.pallas import tpu_sc as plsc

_N = 50000
_E = 1600000
_H = 4
_C1 = 10
_HID = 128
_C2 = _HID // _H
_NC = 5

_NW = 32
_EPW = _E // _NW
_EK = 80
_NCH = _EPW // _EK
_MK = 128
_MCH = 390
_MEXTRA = _NW * _MCH * _MK

_mesh = plsc.VectorSubcoreMesh(core_axis_name="c", subcore_axis_name="s")
_sc_params = pltpu.CompilerParams(needs_layout_passes=False,
                                  use_tc_tiling_on_sc=False)


def _lrelu(v):
    return jnp.where(v >= 0, v, 0.2 * v)


def _msg8_body(dst_hbm, srcblk_hbm, adst_hbm, m_hbm, g_hbm, zero_hbm,
               out_hbm, adst_v, m_v, dst_a, dst_b, idx_a, idx_b, grow_a,
               grow_b, rows_a, rows_b, acc_sh, sem_a0, sem_a1, sem_b0,
               sem_b1):
    cid = lax.axis_index("c")
    sid = lax.axis_index("s")
    wid = sid * 2 + cid
    base = wid * (_MCH * _MK)
    lane = jnp.arange(16, dtype=jnp.int32)
    col_as = jnp.full((16,), 8, jnp.int32)
    dst_s = (dst_a, dst_b)
    idx_s = (idx_a, idx_b)
    grow_s = (grow_a, grow_b)
    rows_s = (rows_a, rows_b)
    sem_as = (sem_a0, sem_a1)
    sem_bs = (sem_b0, sem_b1)

    def phase(ph, carry):
        h = ph // 5
        pltpu.sync_copy(adst_hbm.at[h], adst_v)
        pltpu.sync_copy(m_hbm.at[h], m_v)

        @pl.when(sid == 0)
        def _():
            pltpu.sync_copy(zero_hbm, acc_sh)

        plsc.subcore_barrier()

        def start_a(i, b):
            off = pl.multiple_of(base + i * _MK, 8)
            pltpu.async_copy(dst_hbm.at[pl.ds(off, _MK)], dst_s[b], sem_as[b])
            pltpu.async_copy(srcblk_hbm.at[ph, pl.ds(off, _MK)], idx_s[b],
                             sem_as[b])

        def wait_a(b):
            pltpu.make_async_copy(dst_hbm.at[pl.ds(0, _MK)], dst_s[b],
                                  sem_as[b]).wait()
            pltpu.make_async_copy(dst_hbm.at[pl.ds(0, _MK)], idx_s[b],
                                  sem_as[b]).wait()

        def start_g(b):
            pltpu.async_copy(g_hbm.at[idx_s[b]], grow_s[b], sem_bs[b])

        def wait_g(b):
            pltpu.make_async_copy(g_hbm.at[pl.ds(0, _MK)], grow_s[b],
                                  sem_bs[b]).wait()

        def compute_scatter(b):
            mv = m_v[...]
            for g in range(_MK // 16):
                d16 = dst_s[b][pl.ds(g * 16, 16)]
                l16 = g * 16 + lane
                a = (plsc.load_gather(grow_s[b], [l16, col_as])
                     + plsc.load_gather(adst_v, [d16]))
                ex = jnp.exp(_lrelu(a) - mv)
                for c in range(8):
                    cc = jnp.full((16,), c, jnp.int32)
                    gv = plsc.load_gather(grow_s[b], [l16, cc])
                    plsc.store_scatter(rows_s[b], [l16, cc], ex * gv)
            pltpu.sync_copy(rows_s[b], acc_sh.at[dst_s[b]], add=True)

        start_a(0, 0)
        wait_a(0)
        start_g(0)
        start_a(1, 1)

        def step(i, b):
            @pl.when(i + 1 < _MCH)
            def _():
                wait_a(1 - b)
                start_g(1 - b)
            wait_g(b)
            compute_scatter(b)

            @pl.when(i + 2 < _MCH)
            def _():
                start_a(i + 2, b)

        def pair(io, c2):
            step(io * 2, 0)
            step(io * 2 + 1, 1)
            return c2

        lax.fori_loop(0, _MCH // 2, pair, 0)

        @pl.when(wid < 20)
        def _():
            offx = pl.multiple_of(_MEXTRA + wid * _MK, 8)
            pltpu.async_copy(dst_hbm.at[pl.ds(offx, _MK)], dst_a, sem_a0)
            pltpu.async_copy(srcblk_hbm.at[ph, pl.ds(offx, _MK)], idx_a,
                             sem_a0)
            wait_a(0)
            start_g(0)
            wait_g(0)
            compute_scatter(0)

        plsc.subcore_barrier()

        @pl.when(sid == 0)
        def _():
            pltpu.sync_copy(acc_sh, out_hbm.at[cid, ph])

        return carry

    lax.fori_loop(0, 20, phase, 0)


_msg8_pass = functools.partial(
    pl.kernel,
    _msg8_body,
    out_type=jax.ShapeDtypeStruct((2, 20, _N, 8), jnp.float32),
    mesh=_mesh,
    scratch_types=[
        pltpu.VMEM((_N,), jnp.float32),
        pltpu.VMEM((16,), jnp.float32),
        pltpu.VMEM((_MK,), jnp.int32),
        pltpu.VMEM((_MK,), jnp.int32),
        pltpu.VMEM((_MK,), jnp.int32),
        pltpu.VMEM((_MK,), jnp.int32),
        pltpu.VMEM((_MK, 16), jnp.float32),
        pltpu.VMEM((_MK, 16), jnp.float32),
        pltpu.VMEM((_MK, 8), jnp.float32),
        pltpu.VMEM((_MK, 8), jnp.float32),
        pltpu.VMEM_SHARED((_N, 8), jnp.float32),
        pltpu.SemaphoreType.DMA,
        pltpu.SemaphoreType.DMA,
        pltpu.SemaphoreType.DMA,
        pltpu.SemaphoreType.DMA,
    ],
    compiler_params=_sc_params,
)()


def _branch3_body(src_hbm, dst_hbm, x_hbm, c_hbm, ea0_hbm, ea1_hbm, ea2_hbm,
                  m_hbm, zero_hbm, out_hbm, x_v, c_v, src_v, dst_v, ea0_v,
                  ea1_v, ea2_v, m_v, rows_v, acc_sh):
    cid = lax.axis_index("c")
    sid = lax.axis_index("s")
    wid = sid * 2 + cid
    base = wid * _EPW
    lane = jnp.arange(16, dtype=jnp.int32)

    pltpu.sync_copy(x_hbm, x_v)
    for b in range(3):
        pltpu.sync_copy(c_hbm.at[b], c_v)

        @pl.when(sid == 0)
        def _():
            pltpu.sync_copy(zero_hbm, acc_sh)

        plsc.subcore_barrier()

        def chunk(i, carry):
            off = pl.multiple_of(base + i * _EK, 8)
            pltpu.sync_copy(src_hbm.at[pl.ds(off, _EK)], src_v)
            pltpu.sync_copy(dst_hbm.at[pl.ds(off, _EK)], dst_v)
            pltpu.sync_copy(ea0_hbm.at[pl.ds(off, _EK)], ea0_v)
            pltpu.sync_copy(ea1_hbm.at[pl.ds(off, _EK)], ea1_v)
            pltpu.sync_copy(ea2_hbm.at[pl.ds(off, _EK)], ea2_v)
            pltpu.sync_copy(m_hbm.at[b, pl.ds(off, _EK)], m_v)
            for g in range(_EK // 16):
                sl = pl.ds(g * 16, 16)
                s16 = src_v[sl]
                d16 = dst_v[sl]
                xs = plsc.load_gather(x_v, [s16])
                xd = plsc.load_gather(x_v, [d16])
                e0 = ea0_v[sl]
                e1 = ea1_v[sl]
                e2 = ea2_v[sl]
                m = m_v[sl]
                l16 = g * 16 + lane
                for h in range(4):
                    ws = c_v[pl.ds(h * 16, 16)]
                    wd = c_v[pl.ds((4 + h) * 16, 16)]
                    mh = c_v[pl.ds((8 + h) * 16, 16)]
                    v0 = c_v[pl.ds((12 + h) * 16, 16)]
                    v1 = c_v[pl.ds((16 + h) * 16, 16)]
                    v2 = c_v[pl.ds((20 + h) * 16, 16)]
                    z = xs * ws + xd * wd + e0 * v0 + e1 * v1 + e2 * v2
                    ex = m * jnp.exp(_lrelu(z) - mh)
                    plsc.store_scatter(
                        rows_v, [l16, jnp.full((16,), h, jnp.int32)], ex)
                    plsc.store_scatter(
                        rows_v, [l16, jnp.full((16,), 4 + h, jnp.int32)], ex * xs)
                plsc.store_scatter(rows_v, [l16, jnp.full((16,), 8, jnp.int32)], m)
                plsc.store_scatter(rows_v, [l16, jnp.full((16,), 9, jnp.int32)], e0 * m)
                plsc.store_scatter(rows_v, [l16, jnp.full((16,), 10, jnp.int32)], e1 * m)
                plsc.store_scatter(rows_v, [l16, jnp.full((16,), 11, jnp.int32)], e2 * m)
            pltpu.sync_copy(rows_v, acc_sh.at[dst_v], add=True)
            return carry

        lax.fori_loop(0, _NCH, chunk, 0)
        plsc.subcore_barrier()

        @pl.when(sid == 0)
        def _():
            pltpu.sync_copy(acc_sh, out_hbm.at[cid, b])


_branch3_pass = functools.partial(
    pl.kernel,
    _branch3_body,
    out_type=jax.ShapeDtypeStruct((2, 3, _N, 12), jnp.float32),
    mesh=_mesh,
    scratch_types=[
        pltpu.VMEM((_N,), jnp.float32),
        pltpu.VMEM((24 * 16,), jnp.float32),
        pltpu.VMEM((_EK,), jnp.int32),
        pltpu.VMEM((_EK,), jnp.int32),
        pltpu.VMEM((_EK,), jnp.float32),
        pltpu.VMEM((_EK,), jnp.float32),
        pltpu.VMEM((_EK,), jnp.float32),
        pltpu.VMEM((_EK,), jnp.float32),
        pltpu.VMEM((_EK, 12), jnp.float32),
        pltpu.VMEM_SHARED((_N, 12), jnp.float32),
    ],
    compiler_params=_sc_params,
)()


def _gproj_body(h_ref, w_ref, as_ref, ad_ref, g_ref, asrc_ref, adst_ref):
    g = jnp.dot(h_ref[...], w_ref[...], preferred_element_type=jnp.float32)
    g_ref[...] = g
    asrc_ref[...] = jnp.dot(g, as_ref[...], preferred_element_type=jnp.float32)
    adst_ref[...] = jnp.dot(g, ad_ref[...], preferred_element_type=jnp.float32)


def _gproj(h, W, As, Ad):
    blk = 1000
    return pl.pallas_call(
        _gproj_body,
        grid=(_N // blk,),
        in_specs=[
            pl.BlockSpec((blk, _HID), lambda i: (i, 0)),
            pl.BlockSpec((_HID, _HID), lambda i: (0, 0)),
            pl.BlockSpec((_HID, _H), lambda i: (0, 0)),
            pl.BlockSpec((_HID, _H), lambda i: (0, 0)),
        ],
        out_specs=[
            pl.BlockSpec((blk, _HID), lambda i: (i, 0)),
            pl.BlockSpec((blk, _H), lambda i: (i, 0)),
            pl.BlockSpec((blk, _H), lambda i: (i, 0)),
        ],
        out_shape=[
            jax.ShapeDtypeStruct((_N, _HID), jnp.float32),
            jax.ShapeDtypeStruct((_N, _H), jnp.float32),
            jax.ShapeDtypeStruct((_N, _H), jnp.float32),
        ],
    )(h, W, As, Ad)


def _fuse_body(a_ref, b_ref, c_ref, w1_ref, w2_ref, w3_ref, bias_ref, o_ref):
    z = (jnp.dot(a_ref[...], w1_ref[...], preferred_element_type=jnp.float32)
         + jnp.dot(b_ref[...], w2_ref[...], preferred_element_type=jnp.float32)
         + jnp.dot(c_ref[...], w3_ref[...], preferred_element_type=jnp.float32)
         + bias_ref[...])
    o_ref[...] = jnp.maximum(z, 0.0)


def _fuse(a, b, c, W_fus, b_fus):
    blk = 1000
    hc = _H * _C1
    return pl.pallas_call(
        _fuse_body,
        grid=(_N // blk,),
        in_specs=[
            pl.BlockSpec((blk, hc), lambda i: (i, 0)),
            pl.BlockSpec((blk, hc), lambda i: (i, 0)),
            pl.BlockSpec((blk, hc), lambda i: (i, 0)),
            pl.BlockSpec((hc, _HID), lambda i: (0, 0)),
            pl.BlockSpec((hc, _HID), lambda i: (0, 0)),
            pl.BlockSpec((hc, _HID), lambda i: (0, 0)),
            pl.BlockSpec((1, _HID), lambda i: (0, 0)),
        ],
        out_specs=pl.BlockSpec((blk, _HID), lambda i: (i, 0)),
        out_shape=jax.ShapeDtypeStruct((_N, _HID), jnp.float32),
    )(a, b, c, W_fus[:hc], W_fus[hc:2 * hc], W_fus[2 * hc:], b_fus.reshape(1, -1))


def _mlp_body(h_ref, w1_ref, b1_ref, w2_ref, b2_ref, o_ref):
    h = h_ref[...]
    z = jnp.maximum(jnp.dot(h, w1_ref[...], preferred_element_type=jnp.float32)
                    + b1_ref[...], 0.0)
    lg = jnp.dot(z, w2_ref[...], preferred_element_type=jnp.float32) + b2_ref[...]
    m = jnp.max(lg, axis=-1, keepdims=True)
    e = jnp.exp(lg - m)
    o_ref[...] = lg - m - jnp.log(jnp.sum(e, axis=-1, keepdims=True))


def _mlp_head(h, W_c1, b_c1, W_c2, b_c2):
    blk = 1000
    return pl.pallas_call(
        _mlp_body,
        grid=(_N // blk,),
        in_specs=[
            pl.BlockSpec((blk, _HID), lambda i: (i, 0)),
            pl.BlockSpec((_HID, _HID // 2), lambda i: (0, 0)),
            pl.BlockSpec((1, _HID // 2), lambda i: (0, 0)),
            pl.BlockSpec((_HID // 2, _NC), lambda i: (0, 0)),
            pl.BlockSpec((1, _NC), lambda i: (0, 0)),
        ],
        out_specs=pl.BlockSpec((blk, _NC), lambda i: (i, 0)),
        out_shape=jax.ShapeDtypeStruct((_N, _NC), jnp.float32),
    )(h, W_c1, b_c1.reshape(1, -1), W_c2, b_c2.reshape(1, -1))


def _branch_params(stats, W, att_s, att_d, W_e, att_e):
    xmax, xmin, eamax, eamin = stats
    Wm = W.reshape(_H, _C1)
    ws = (Wm * att_s).sum(-1)
    wd = (Wm * att_d).sum(-1)
    v = (W_e.reshape(3, _H, _C1) * att_e[None]).sum(-1)
    ms = jnp.maximum(xmax * ws, xmin * ws)
    md = jnp.maximum(xmax * wd, xmin * wd)
    aeb = jnp.maximum(eamax[:, None] * v, eamin[:, None] * v).sum(0)
    M = _lrelu(ms + md + aeb)
    consts = jnp.concatenate([ws, wd, M, v.reshape(12)])
    c_arr = jnp.tile(consts[:, None], (1, 16)).reshape(-1)
    return ws, wd, v, M, c_arr


def _branch_finish(xf, tot, ws, wd, v, M, W, b):
    den = tot[:, 0:4]
    num = tot[:, 4:8]
    cnt = tot[:, 8]
    eas = tot[:, 9:12]
    loop_attr = eas / jnp.maximum(cnt, 1.0)[:, None]
    alpha_loop = _lrelu(xf[:, None] * (ws + wd) + loop_attr @ v)
    exloop = jnp.exp(alpha_loop - M)
    coef = (num + exloop * xf[:, None]) / (den + exloop)
    out = coef[:, :, None] * W.reshape(_H, _C1)[None]
    return out.reshape(_N, _H * _C1) + b


def _gat_dense(h, src, dst, srcblk, zero8, W, att_s, att_d, b):
    emb = jnp.zeros((_HID, _H), jnp.float32)
    rows = jnp.arange(_HID)
    hsel = jnp.repeat(jnp.arange(_H), _C2)
    As = emb.at[rows, hsel].set(att_s.reshape(-1))
    Ad = emb.at[rows, hsel].set(att_d.reshape(-1))
    g, asrc, adst = _gproj(h, W, As, Ad)
    M = _lrelu(jnp.max(asrc, axis=0) + jnp.max(adst, axis=0))
    g4 = g.reshape(_N, 4, 32)
    blocks = jnp.concatenate([g4, jnp.ones((_N, 4, 1), jnp.float32),
                              jnp.zeros((_N, 4, 7), jnp.float32)],
                             axis=2).reshape(_N, 4, 5, 8)
    acol = jnp.broadcast_to(asrc[:, :, None, None], (_N, 4, 5, 1))
    gx = jnp.concatenate([blocks, acol,
                          jnp.zeros((_N, 4, 5, 7), jnp.float32)], axis=3)
    g_flat = jnp.transpose(gx, (1, 2, 0, 3)).reshape(20 * _N, 16)
    m4 = jnp.tile(M[:, None], (1, 16))
    parts = _msg8_pass(dst, srcblk, adst.T, m4, g_flat, zero8)
    tot = (parts[0] + parts[1]).reshape(4, 5, _N, 8)
    msg = jnp.transpose(tot[:, :4], (2, 0, 1, 3)).reshape(_N, _H, _C2)
    den = tot[:, 4, :, 0].T
    exloop = jnp.exp(_lrelu(asrc + adst) - M)
    g3 = g.reshape(_N, _H, _C2)
    out = (msg + exloop[:, :, None] * g3) / (den + exloop)[:, :, None]
    return out.reshape(_N, _HID) + b


def kernel(x, edge_index, edge_attr, direct_temporal_mask, gap_temporal_mask,
           proximity_mask, W_t, att_src_t, att_dst_t, W_e_t, att_e_t, b_t,
           W_g, att_src_g, att_dst_g, W_e_g, att_e_g, b_g,
           W_p, att_src_p, att_dst_p, W_e_p, att_e_p, b_p,
           W_fus, b_fus, W2, att_src2, att_dst2, b2,
           W3, att_src3, att_dst3, b3, W_c1, b_c1, W_c2, b_c2):
    src = edge_index[0]
    dst = edge_index[1]
    xf = x[:, 0]
    eaT = edge_attr.T
    ea0, ea1, ea2 = eaT[0], eaT[1], eaT[2]
    masks = jnp.stack([direct_temporal_mask.astype(jnp.float32),
                       gap_temporal_mask.astype(jnp.float32),
                       proximity_mask.astype(jnp.float32)])
    zero12 = jnp.zeros((_N, 12), jnp.float32)
    zero8 = jnp.zeros((_N, 8), jnp.float32)
    stats = (jnp.max(xf), jnp.min(xf),
             jnp.max(edge_attr, axis=0), jnp.min(edge_attr, axis=0))
    p_t = _branch_params(stats, W_t, att_src_t, att_dst_t, W_e_t, att_e_t)
    p_g = _branch_params(stats, W_g, att_src_g, att_dst_g, W_e_g, att_e_g)
    p_p = _branch_params(stats, W_p, att_src_p, att_dst_p, W_e_p, att_e_p)
    c_all = jnp.stack([p_t[4], p_g[4], p_p[4]])
    parts = _branch3_pass(src, dst, xf, c_all, ea0, ea1, ea2, masks, zero12)
    tots = parts[0] + parts[1]
    h_t = _branch_finish(xf, tots[0], *p_t[:4], W_t, b_t)
    h_g = _branch_finish(xf, tots[1], *p_g[:4], W_g, b_g)
    h_p = _branch_finish(xf, tots[2], *p_p[:4], W_p, b_p)
    h = _fuse(h_t, h_g, h_p, W_fus, b_fus)
    srcblk = src[None, :] + (jnp.arange(20, dtype=jnp.int32) * _N)[:, None]
    h = jax.nn.relu(_gat_dense(h, src, dst, srcblk, zero8,
                               W2, att_src2, att_dst2, b2))
    h = jax.nn.relu(_gat_dense(h, src, dst, srcblk, zero8,
                               W3, att_src3, att_dst3, b3))
    return _mlp_head(h, W_c1, b_c1, W_c2, b_c2)
